# Initial kernel scaffold; baseline (speedup 1.0000x reference)
import jax
import jax.numpy as jnp
from jax.experimental import pallas as pl


def kernel(x, edge_attr, params, edge_index, node_types):
    raise NotImplementedError("write your pallas kernel here")


# speedup vs baseline: 2.2900x; 2.2900x over previous
import functools

import jax
import jax.numpy as jnp
from jax import lax
from jax.experimental import pallas as pl
from jax.experimental.pallas import tpu as pltpu
from jax.experimental.pallas import tpu_sc as plsc

N = 10000
E = 320000
DF = 128
DE = 128
DH = 256
NPAD = 10112
TRASH = N

_TYPE_MAP_TUPLE = (0, 0, 0, 0, 0, 1, 2, 1, 2, 1, 2, 3, 4, 3, 4, 5, 5)

NC, NS, L = 2, 16, 16
NW = NC * NS


_EPW = E // NW
_EP_CHUNK = 2000


def _cat16(t):
  par = lax.rem(t, 2)
  c_mid = 2 - par
  c_hi = 4 - par
  return jnp.where(t < 5, 0,
                   jnp.where(t < 11, c_mid, jnp.where(t < 15, c_hi, 5)))


def _edge_prep_body(src_hbm, dst_hbm, ntypes_hbm,
                    maskf_hbm, idxm_hbm, idxnm_hbm,
                    src_v, dst_v, ts_v, td_v, mf_v, im_v, inm_v, sem):
  wid = lax.axis_index("s") * NC + lax.axis_index("c")
  base = wid * _EPW

  def chunk_body(ci, _):
    off = base + ci * _EP_CHUNK
    pltpu.sync_copy(src_hbm.at[pl.ds(off, _EP_CHUNK)], src_v)
    pltpu.sync_copy(dst_hbm.at[pl.ds(off, _EP_CHUNK)], dst_v)
    pltpu.async_copy(ntypes_hbm.at[src_v], ts_v, sem).wait()
    pltpu.async_copy(ntypes_hbm.at[dst_v], td_v, sem).wait()

    def vec_body(i, _):
      sl = pl.ds(i * L, L)
      m = _cat16(ts_v[sl]) == _cat16(td_v[sl])
      dv = dst_v[sl]
      mf_v[sl] = jnp.where(m, 1.0, 0.0).astype(jnp.float32)
      im_v[sl] = jnp.where(m, dv, TRASH).astype(jnp.int32)
      inm_v[sl] = jnp.where(m, TRASH, dv).astype(jnp.int32)
      return 0

    lax.fori_loop(0, _EP_CHUNK // L, vec_body, 0)
    pltpu.sync_copy(mf_v, maskf_hbm.at[pl.ds(off, _EP_CHUNK)])
    pltpu.sync_copy(im_v, idxm_hbm.at[pl.ds(off, _EP_CHUNK)])
    pltpu.sync_copy(inm_v, idxnm_hbm.at[pl.ds(off, _EP_CHUNK)])
    return 0

  lax.fori_loop(0, _EPW // _EP_CHUNK, chunk_body, 0)


def _edge_prep(src, dst, node_types):
  mesh = plsc.VectorSubcoreMesh(core_axis_name="c", subcore_axis_name="s")
  k = pl.kernel(
      _edge_prep_body,
      mesh=mesh,
      out_type=(
          jax.ShapeDtypeStruct((E,), jnp.float32),
          jax.ShapeDtypeStruct((E,), jnp.int32),
          jax.ShapeDtypeStruct((E,), jnp.int32),
      ),
      scratch_types=[
          pltpu.VMEM((_EP_CHUNK,), jnp.int32),
          pltpu.VMEM((_EP_CHUNK,), jnp.int32),
          pltpu.VMEM((_EP_CHUNK,), jnp.int32),
          pltpu.VMEM((_EP_CHUNK,), jnp.int32),
          pltpu.VMEM((_EP_CHUNK,), jnp.float32),
          pltpu.VMEM((_EP_CHUNK,), jnp.int32),
          pltpu.VMEM((_EP_CHUNK,), jnp.int32),
          pltpu.SemaphoreType.DMA,
      ],
  )
  return k(src, dst, node_types)


_GPW = E // NW
_G_CHUNK = 200


def _gather_body(table_hbm, idxa_hbm, idxb_hbm, outa_hbm, outb_hbm,
                 idx_v, rows_v, sem):
  wid = lax.axis_index("s") * NC + lax.axis_index("c")
  base = wid * _GPW

  def chunk_body(ci, _):
    off = base + ci * _G_CHUNK
    pltpu.sync_copy(idxa_hbm.at[pl.ds(off, _G_CHUNK)], idx_v)
    pltpu.async_copy(table_hbm.at[idx_v], rows_v, sem).wait()
    pltpu.sync_copy(rows_v, outa_hbm.at[pl.ds(off, _G_CHUNK)])
    pltpu.sync_copy(idxb_hbm.at[pl.ds(off, _G_CHUNK)], idx_v)
    pltpu.async_copy(table_hbm.at[idx_v], rows_v, sem).wait()
    pltpu.sync_copy(rows_v, outb_hbm.at[pl.ds(off, _G_CHUNK)])
    return 0

  lax.fori_loop(0, _GPW // _G_CHUNK, chunk_body, 0)


def _gather_rows(table, idxa, idxb):
  mesh = plsc.VectorSubcoreMesh(core_axis_name="c", subcore_axis_name="s")
  k = pl.kernel(
      _gather_body,
      mesh=mesh,
      out_type=(
          jax.ShapeDtypeStruct((E, DF), jnp.float32),
          jax.ShapeDtypeStruct((E, DF), jnp.float32),
      ),
      scratch_types=[
          pltpu.VMEM((_G_CHUNK,), jnp.int32),
          pltpu.VMEM((_G_CHUNK, DF), jnp.float32),
          pltpu.SemaphoreType.DMA,
      ],
  )
  return k(table, idxa, idxb)


_SPW = E // NW
_S_CHUNK = 200
_ROWS_PER_SUB = NPAD // NS


def _scatter_body(rows_hbm, idx_hbm, out_hbm, idx_v, rows_v, acc_sh):
  cid = lax.axis_index("c")
  sid = lax.axis_index("s")
  wid = sid * NC + cid

  rows_v[...] = jnp.zeros_like(rows_v)
  zbase = sid * _ROWS_PER_SUB
  for zoff in (0, 200, 400):
    pltpu.sync_copy(rows_v, acc_sh.at[pl.ds(zbase + zoff, _S_CHUNK)])
  pltpu.sync_copy(rows_v.at[pl.ds(0, 32)], acc_sh.at[pl.ds(zbase + 600, 32)])
  plsc.subcore_barrier()

  base = wid * _SPW

  def chunk_body(ci, _):
    off = base + ci * _S_CHUNK
    pltpu.sync_copy(idx_hbm.at[pl.ds(off, _S_CHUNK)], idx_v)
    pltpu.sync_copy(rows_hbm.at[pl.ds(off, _S_CHUNK)], rows_v)
    pltpu.sync_copy(rows_v, acc_sh.at[idx_v], add=True)
    return 0

  lax.fori_loop(0, _SPW // _S_CHUNK, chunk_body, 0)
  plsc.subcore_barrier()

  pltpu.sync_copy(acc_sh.at[pl.ds(sid * _ROWS_PER_SUB, _ROWS_PER_SUB)],
                  out_hbm.at[cid, pl.ds(sid * _ROWS_PER_SUB, _ROWS_PER_SUB)])


def _segment_sum(rows, idx):
  mesh = plsc.VectorSubcoreMesh(core_axis_name="c", subcore_axis_name="s")
  k = pl.kernel(
      _scatter_body,
      mesh=mesh,
      out_type=jax.ShapeDtypeStruct((NC, NPAD, DE), jnp.float32),
      scratch_types=[
          pltpu.VMEM((_S_CHUNK,), jnp.int32),
          pltpu.VMEM((_S_CHUNK, DE), jnp.float32),
          pltpu.VMEM_SHARED((NPAD, DE), jnp.float32),
      ],
  )
  return k(rows, idx)


_BN = 400
_BE = 512


def _dot(a, b):
  return jax.lax.dot_general(a, b, (((1,), (0,)), ((), ())),
                             preferred_element_type=jnp.float32)


def _node_enc_body(x_ref, w0_ref, b0_ref, w1_ref, b1_ref, o_ref):
  h = jnp.maximum(_dot(x_ref[...], w0_ref[...]) + b0_ref[...], 0.0)
  o_ref[...] = _dot(h, w1_ref[...]) + b1_ref[...]


def _mlp2_rows(x, w0, b0, w1, b1, block):
  rows = x.shape[0]
  d_in, d_h = w0.shape
  d_out = w1.shape[1]
  return pl.pallas_call(
      _node_enc_body,
      grid=(rows // block,),
      in_specs=[
          pl.BlockSpec((block, d_in), lambda i: (i, 0)),
          pl.BlockSpec((d_in, d_h), lambda i: (0, 0)),
          pl.BlockSpec((1, d_h), lambda i: (0, 0)),
          pl.BlockSpec((d_h, d_out), lambda i: (0, 0)),
          pl.BlockSpec((1, d_out), lambda i: (0, 0)),
      ],
      out_specs=pl.BlockSpec((block, d_out), lambda i: (i, 0)),
      out_shape=jax.ShapeDtypeStruct((rows, d_out), jnp.float32),
  )(x, w0, b0[None, :], w1, b1[None, :])


def _edge_mlp_sel_body(gs_ref, gd_ref, ea_ref, eb_ref, m_ref,
                       w1a_ref, w1b_ref, w1c_ref, b1_ref, w2_ref, b2_ref,
                       o_ref):
  m = m_ref[...]
  ef = ea_ref[...] * m + eb_ref[...] * (1.0 - m)
  h = _dot(gs_ref[...], w1a_ref[...])
  h += _dot(gd_ref[...], w1b_ref[...])
  h += _dot(ef, w1c_ref[...])
  h = jnp.maximum(h + b1_ref[...], 0.0)
  o_ref[...] = jnp.maximum(_dot(h, w2_ref[...]) + b2_ref[...], 0.0)


def _edge_mlp_body(gs_ref, gd_ref, ef_ref, m_ref,
                   w1a_ref, w1b_ref, w1c_ref, b1_ref, w2_ref, b2_ref,
                   o_ref):
  h = _dot(gs_ref[...], w1a_ref[...])
  h += _dot(gd_ref[...], w1b_ref[...])
  h += _dot(ef_ref[...], w1c_ref[...])
  h = jnp.maximum(h + b1_ref[...], 0.0)
  o_ref[...] = jnp.maximum(_dot(h, w2_ref[...]) + b2_ref[...], 0.0)


def _edge_mlp(gs, gd, efa, efb, maskf, p):
  w1a = p['Wm1'][:DF]
  w1b = p['Wm1'][DF:2 * DF]
  w1c = p['Wm1'][2 * DF:]
  weight_specs = [
      pl.BlockSpec((DF, DH), lambda i: (0, 0)),
      pl.BlockSpec((DF, DH), lambda i: (0, 0)),
      pl.BlockSpec((DE, DH), lambda i: (0, 0)),
      pl.BlockSpec((1, DH), lambda i: (0, 0)),
      pl.BlockSpec((DH, DE), lambda i: (0, 0)),
      pl.BlockSpec((1, DE), lambda i: (0, 0)),
  ]
  row_spec = pl.BlockSpec((_BE, DF), lambda i: (i, 0))
  if efb is None:
    return pl.pallas_call(
        _edge_mlp_body,
        grid=(E // _BE,),
        in_specs=[row_spec, row_spec, row_spec,
                  pl.BlockSpec((_BE, 1), lambda i: (i, 0))] + weight_specs,
        out_specs=pl.BlockSpec((_BE, DE), lambda i: (i, 0)),
        out_shape=jax.ShapeDtypeStruct((E, DE), jnp.float32),
    )(gs, gd, efa, maskf, w1a, w1b, w1c, p['bm1'][None, :], p['Wm2'],
      p['bm2'][None, :])
  return pl.pallas_call(
      _edge_mlp_sel_body,
      grid=(E // _BE,),
      in_specs=[row_spec, row_spec, row_spec, row_spec,
                pl.BlockSpec((_BE, 1), lambda i: (i, 0))] + weight_specs,
      out_specs=pl.BlockSpec((_BE, DE), lambda i: (i, 0)),
      out_shape=jax.ShapeDtypeStruct((E, DE), jnp.float32),
  )(gs, gd, efa, efb, maskf, w1a, w1b, w1c, p['bm1'][None, :], p['Wm2'],
    p['bm2'][None, :])


def _node_update_body(nf_ref, agg_ref, wa_ref, wb_ref, b_ref, o_ref):
  agg = agg_ref[0] + agg_ref[1]
  h = _dot(nf_ref[...], wa_ref[...]) + _dot(agg, wb_ref[...]) + b_ref[...]
  o_ref[...] = jnp.maximum(h, 0.0)


def _node_update(nf, agg2, p):
  wa = p['Wnu'][:DF]
  wb = p['Wnu'][DF:]
  return pl.pallas_call(
      _node_update_body,
      grid=(N // _BN,),
      in_specs=[
          pl.BlockSpec((_BN, DF), lambda i: (i, 0)),
          pl.BlockSpec((NC, _BN, DE), lambda i: (0, i, 0)),
          pl.BlockSpec((DF, DF), lambda i: (0, 0)),
          pl.BlockSpec((DE, DF), lambda i: (0, 0)),
          pl.BlockSpec((1, DF), lambda i: (0, 0)),
      ],
      out_specs=pl.BlockSpec((_BN, DF), lambda i: (i, 0)),
      out_shape=jax.ShapeDtypeStruct((N, DF), jnp.float32),
  )(nf, agg2, wa, wb, p['bnu'][None, :])


def _edge_head_body(ea_ref, eb_ref, m_ref, w0_ref, b0_ref, w1_ref, b1_ref,
                    o_ref):
  m = m_ref[...]
  ef = ea_ref[...] * m + eb_ref[...] * (1.0 - m)
  h = jnp.maximum(_dot(ef, w0_ref[...]) + b0_ref[...], 0.0)
  o_ref[...] = _dot(h, w1_ref[...]) + b1_ref[...]


def _edge_head(ea, eb, maskf, p):
  return pl.pallas_call(
      _edge_head_body,
      grid=(E // _BE,),
      in_specs=[
          pl.BlockSpec((_BE, DE), lambda i: (i, 0)),
          pl.BlockSpec((_BE, DE), lambda i: (i, 0)),
          pl.BlockSpec((_BE, 1), lambda i: (i, 0)),
          pl.BlockSpec((DE, 64), lambda i: (0, 0)),
          pl.BlockSpec((1, 64), lambda i: (0, 0)),
          pl.BlockSpec((64, 1), lambda i: (0, 0)),
          pl.BlockSpec((1, 1), lambda i: (0, 0)),
      ],
      out_specs=pl.BlockSpec((_BE, 1), lambda i: (i, 0)),
      out_shape=jax.ShapeDtypeStruct((E, 1), jnp.float32),
  )(ea, eb, maskf, p['Wec0'], p['bec0'][None, :], p['Wec1'],
    p['bec1'][None, :])


def _node_heads_body(nf_ref, wn0_ref, bn0_ref, wn1_ref, bn1_ref,
                     wc0_ref, bc0_ref, wc1_ref, bc1_ref, on_ref, oc_ref):
  nf = nf_ref[...]
  h1 = jnp.maximum(_dot(nf, wn0_ref[...]) + bn0_ref[...], 0.0)
  on_ref[...] = _dot(h1, wn1_ref[...]) + bn1_ref[...]
  h2 = jnp.maximum(_dot(nf, wc0_ref[...]) + bc0_ref[...], 0.0)
  oc_ref[...] = _dot(h2, wc1_ref[...]) + bc1_ref[...]


def _node_heads(nf, p):
  return pl.pallas_call(
      _node_heads_body,
      grid=(N // _BN,),
      in_specs=[
          pl.BlockSpec((_BN, DF), lambda i: (i, 0)),
          pl.BlockSpec((DF, 64), lambda i: (0, 0)),
          pl.BlockSpec((1, 64), lambda i: (0, 0)),
          pl.BlockSpec((64, 1), lambda i: (0, 0)),
          pl.BlockSpec((1, 1), lambda i: (0, 0)),
          pl.BlockSpec((DF, 64), lambda i: (0, 0)),
          pl.BlockSpec((1, 64), lambda i: (0, 0)),
          pl.BlockSpec((64, 6), lambda i: (0, 0)),
          pl.BlockSpec((1, 6), lambda i: (0, 0)),
      ],
      out_specs=[
          pl.BlockSpec((_BN, 1), lambda i: (i, 0)),
          pl.BlockSpec((_BN, 6), lambda i: (i, 0)),
      ],
      out_shape=[
          jax.ShapeDtypeStruct((N, 1), jnp.float32),
          jax.ShapeDtypeStruct((N, 6), jnp.float32),
      ],
  )(nf, p['Wnc0'], p['bnc0'][None, :], p['Wnc1'], p['bnc1'][None, :],
    p['Wc0'], p['bc0'][None, :], p['Wc1'], p['bc1'][None, :])


def _layer(nf, efa, efb, maskf, idx_scatter, src, dst, p):
  gs, gd = _gather_rows(nf, src, dst)
  new_e = _edge_mlp(gs, gd, efa, efb, maskf, p)
  agg2 = _segment_sum(new_e, idx_scatter)
  nf_new = _node_update(nf, agg2, p)
  return nf_new, new_e


def kernel(x, edge_attr, params, edge_index, node_types):
  p = params
  src = edge_index[0]
  dst = edge_index[1]
  maskf, idxm, idxnm = _edge_prep(src, dst, node_types)
  maskf2 = maskf[:, None]

  nf = _mlp2_rows(x, p['Wn0'], p['bn0'], p['Wn1'], p['bn1'], _BN)
  ef = _mlp2_rows(edge_attr, p['We0'], p['be0'], p['We1'], p['be1'], _BE)

  nf, e1 = _layer(nf, ef, None, maskf2, idxm, src, dst, p)
  nf, e2 = _layer(nf, ef, None, maskf2, idxnm, src, dst, p)
  nf, e3 = _layer(nf, e1, e2, maskf2, idxm, src, dst, p)
  nf, e4 = _layer(nf, e1, e2, maskf2, idxnm, src, dst, p)

  pred_edge = _edge_head(e3, e4, maskf2, p)[:, 0]

  nf, ef = _layer(nf, e3, e4, maskf2, dst, src, dst, p)
  nf, ef = _layer(nf, ef, None, maskf2, dst, src, dst, p)

  pred_node, pred_cls = _node_heads(nf, p)
  return (pred_edge, pred_node[:, 0], pred_cls)
